# baseline (speedup 1.0000x reference)
import functools

import jax
import jax.numpy as jnp
from jax import lax
from jax.experimental import pallas as pl
from jax.experimental.pallas import tpu as pltpu

N = 10000
D_IN = 128
HID = 32
D_OUT = 16

_DNT = (((0,), (1,)), ((), ()))


def _mlp(x_ref, wp_ref, b2_ref, o_hbm, ov, out_sem):
    w0 = wp_ref[0:D_IN, :]
    w1 = wp_ref[D_IN:D_IN + HID, :]
    w2 = wp_ref[D_IN + HID:D_IN + 2 * HID, 0:D_OUT]
    b0 = wp_ref[D_IN + 2 * HID:D_IN + 2 * HID + 1, :]
    b1 = wp_ref[D_IN + 2 * HID + 1:D_IN + 2 * HID + 2, :]
    h = jnp.dot(x_ref[...], w0, preferred_element_type=jnp.float32)
    h = jnp.maximum(h + b0, 0.0)
    h = jnp.dot(h, w1, preferred_element_type=jnp.float32)
    h = jnp.maximum(h + b1, 0.0)
    ot = lax.dot_general(w2, h, _DNT,
                         preferred_element_type=jnp.float32)
    ov[...] = ot + b2_ref[...]
    pltpu.make_async_copy(ov, o_hbm, out_sem).start()
    pltpu.make_async_copy(ov, o_hbm, out_sem).wait()


@functools.partial(jax.jit, static_argnames=())
def kernel(x, weight, W0, b0, W1, b1, W2, b2, edge_index, batch):
    del weight, edge_index, batch
    b2c = b2.reshape(D_OUT, 1)
    wp = jnp.concatenate([
        W0,
        W1,
        jnp.pad(W2, ((0, 0), (0, HID - D_OUT))),
        b0.reshape(1, HID),
        b1.reshape(1, HID),
        jnp.zeros((6, HID), jnp.float32),
    ], axis=0)
    vmem = pl.BlockSpec(memory_space=pltpu.MemorySpace.VMEM)
    packed = pl.pallas_call(
        _mlp,
        in_specs=[vmem] * 3,
        out_specs=pl.BlockSpec(memory_space=pl.ANY),
        out_shape=jax.ShapeDtypeStruct((D_OUT, N), jnp.float32),
        scratch_shapes=[
            pltpu.VMEM((D_OUT, N), jnp.float32),
            pltpu.SemaphoreType.DMA,
        ],
    )(x, wp, b2c)
    return packed.T
